# baseline (speedup 1.0000x reference)
import functools

import jax
import jax.numpy as jnp
from jax import lax
from jax.experimental import pallas as pl
from jax.experimental.pallas import tpu as pltpu
from jax.experimental.pallas import tpu_sc as plsc

D = 128
H = 128
B = 1024
L = 50

NW = 32
ROWS_W = B * L // NW
CH = 80
NCH = ROWS_W // CH

bf16 = jnp.bfloat16


def _sc_gather(idx, table):
    mesh = plsc.VectorSubcoreMesh(core_axis_name="c", subcore_axis_name="s")

    @functools.partial(
        pl.kernel,
        mesh=mesh,
        out_type=jax.ShapeDtypeStruct((B * L, D), jnp.float32),
        scratch_types=[
            pltpu.VMEM((NCH, CH), jnp.int32),
            pltpu.VMEM((2, CH, D), jnp.float32),
            pltpu.SemaphoreType.DMA,
            pltpu.SemaphoreType.DMA,
        ],
    )
    def k(idx_hbm, table_hbm, out_hbm, idx_v, rows_v, sem0, sem1):
        wid = lax.axis_index("s") * 2 + lax.axis_index("c")
        pltpu.sync_copy(idx_hbm.at[wid], idx_v)
        base = wid * ROWS_W
        pltpu.async_copy(table_hbm.at[idx_v.at[0]], rows_v.at[0], sem0)
        pltpu.async_copy(table_hbm.at[idx_v.at[1]], rows_v.at[1], sem1)

        def pair(jj, carry):
            j0 = 2 * jj
            pltpu.make_async_copy(
                table_hbm.at[idx_v.at[j0]], rows_v.at[0], sem0).wait()
            pltpu.sync_copy(rows_v.at[0],
                            out_hbm.at[pl.ds(base + j0 * CH, CH)])

            @pl.when(jj < NCH // 2 - 1)
            def _():
                pltpu.async_copy(
                    table_hbm.at[idx_v.at[j0 + 2]], rows_v.at[0], sem0)

            pltpu.make_async_copy(
                table_hbm.at[idx_v.at[j0 + 1]], rows_v.at[1], sem1).wait()
            pltpu.sync_copy(rows_v.at[1],
                            out_hbm.at[pl.ds(base + (j0 + 1) * CH, CH)])

            @pl.when(jj < NCH // 2 - 1)
            def _():
                pltpu.async_copy(
                    table_hbm.at[idx_v.at[j0 + 3]], rows_v.at[1], sem1)

            return carry

        lax.fori_loop(0, NCH // 2, pair, 0)

    return k(idx, table)


def _gru_body(emb_ref, lens_ref, wrzf_ref, winf_ref, whnf_ref, brzf_ref,
              binf_ref, bhnf_ref, wrzb_ref, winb_ref, whnb_ref, brzb_ref,
              binb_ref, bhnb_ref, wcomb_ref, bl_ref, out_ref, hfb_ref):
    lens = lens_ref[...]
    linv = 1.0 / lens.astype(jnp.float32)
    half = jnp.bfloat16(0.5)

    def cell(x, h, h16, wrz_ref, win_ref, whn_ref, brz_ref, bin_ref, bhn_ref):
        xb = x.astype(bf16)
        a = (jnp.dot(xb, wrz_ref[:D],
                     preferred_element_type=jnp.float32)
             + jnp.dot(h16, wrz_ref[D:],
                       preferred_element_type=jnp.float32)
             + brz_ref[...])
        hn = jnp.dot(h16, whn_ref[...],
                     preferred_element_type=jnp.float32) + bhn_ref[...]
        i_n = jnp.dot(xb, win_ref[...],
                      preferred_element_type=jnp.float32) + bin_ref[...]
        tr = jnp.tanh(a[:, :H])
        tz = jnp.tanh(a[:, H:])
        n = jnp.tanh(i_n + hn + tr * hn)
        z = half * tz + half
        hnew = n + z * (h - n)
        return hnew, hnew.astype(bf16)

    h0 = jnp.zeros((B, H), jnp.float32)
    h016 = jnp.zeros((B, H), bf16)

    def cells(i, hf, hf16, hb, hb16):
        tb = L - 1 - i
        hf, hf16 = cell(emb_ref[i], hf, hf16, wrzf_ref, winf_ref, whnf_ref,
                        brzf_ref, binf_ref, bhnf_ref)
        hb, hb16 = cell(emb_ref[tb], hb, hb16, wrzb_ref, winb_ref, whnb_ref,
                        brzb_ref, binb_ref, bhnb_ref)
        hfb_ref[i, :, :H] = hf16
        hfb_ref[tb, :, H:] = hb16
        return hf, hf16, hb, hb16

    def comb(t, acc):
        y = jnp.dot(hfb_ref[t], wcomb_ref[...],
                    preferred_element_type=jnp.float32)
        y = jnp.maximum(y + bl_ref[...], 0.0)
        wc = jnp.sum(jnp.where(lens > t, linv, 0.0))
        return acc + wc * jnp.sum(y, axis=0, keepdims=True)

    def step_a(i, carry):
        return cells(i, *carry)

    carry = lax.fori_loop(0, L // 2, step_a, (h0, h016, h0, h016), unroll=25)

    def step_b(i, carry):
        hf, hf16, hb, hb16, acc = carry
        hf, hf16, hb, hb16 = cells(i, hf, hf16, hb, hb16)
        acc = comb(i, acc)
        acc = comb(L - 1 - i, acc)
        return hf, hf16, hb, hb16, acc

    acc0 = jnp.zeros((1, D), jnp.float32)
    carry = lax.fori_loop(L // 2, L, step_b, (*carry, acc0), unroll=25)
    out_ref[...] = carry[-1]


def _tc_gru(emb_lbd, lens_i, *weights):
    return pl.pallas_call(
        _gru_body,
        out_shape=jax.ShapeDtypeStruct((1, D), jnp.float32),
        scratch_shapes=[pltpu.VMEM((L, B, 2 * H), bf16)],
    )(emb_lbd, lens_i, *weights)


def _dir_weights(Wih, Whh, bih, bhh):
    wihT = Wih.T
    whhT = Whh.T
    wrz = (0.5 * jnp.concatenate([wihT[:, :2 * H], whhT[:, :2 * H]],
                                 axis=0)).astype(bf16)
    win = wihT[:, 2 * H:].astype(bf16)
    whn = (0.5 * whhT[:, 2 * H:]).astype(bf16)
    brz = (0.5 * (bih[:2 * H] + bhh[:2 * H])).reshape(1, 2 * H)
    bin_ = bih[2 * H:].reshape(1, H)
    bhn = (0.5 * bhh[2 * H:]).reshape(1, H)
    return wrz, win, whn, brz, bin_, bhn


def kernel(x_in, in_len, table, Wih_f, Whh_f, bih_f, bhh_f,
           Wih_b, Whh_b, bih_b, bhh_b, Wl, bl):
    x_in = x_in.astype(jnp.int32)
    idx = x_in.T.reshape(NW, NCH, CH)
    emb = _sc_gather(idx, table).reshape(L, B, D)

    lens_i = in_len.astype(jnp.int32).reshape(8, 128)
    out = _tc_gru(
        emb, lens_i,
        *_dir_weights(Wih_f, Whh_f, bih_f, bhh_f),
        *_dir_weights(Wih_b, Whh_b, bih_b, bhh_b),
        Wl.T.astype(bf16), bl.reshape(1, D),
    )
    return out


# speedup vs baseline: 1.0673x; 1.0673x over previous
import functools

import jax
import jax.numpy as jnp
from jax import lax
from jax.experimental import pallas as pl
from jax.experimental.pallas import tpu as pltpu
from jax.experimental.pallas import tpu_sc as plsc

D = 128
H = 128
B = 1024
L = 50

NW = 32
ROWS_W = B * L // NW
CH = 80
NCH = ROWS_W // CH

bf16 = jnp.bfloat16


def _sc_gather(idx, table):
    mesh = plsc.VectorSubcoreMesh(core_axis_name="c", subcore_axis_name="s")

    @functools.partial(
        pl.kernel,
        mesh=mesh,
        out_type=jax.ShapeDtypeStruct((B * L, D), jnp.float32),
        scratch_types=[
            pltpu.VMEM((NCH, CH), jnp.int32),
            pltpu.VMEM((2, CH, D), jnp.float32),
            pltpu.SemaphoreType.DMA,
            pltpu.SemaphoreType.DMA,
        ],
    )
    def k(idx_hbm, table_hbm, out_hbm, idx_v, rows_v, sem0, sem1):
        wid = lax.axis_index("s") * 2 + lax.axis_index("c")
        pltpu.sync_copy(idx_hbm.at[wid], idx_v)
        base = wid * ROWS_W
        pltpu.async_copy(table_hbm.at[idx_v.at[0]], rows_v.at[0], sem0)
        pltpu.async_copy(table_hbm.at[idx_v.at[1]], rows_v.at[1], sem1)

        def pair(jj, carry):
            j0 = 2 * jj
            pltpu.make_async_copy(
                table_hbm.at[idx_v.at[j0]], rows_v.at[0], sem0).wait()
            pltpu.sync_copy(rows_v.at[0],
                            out_hbm.at[pl.ds(base + j0 * CH, CH)])

            @pl.when(jj < NCH // 2 - 1)
            def _():
                pltpu.async_copy(
                    table_hbm.at[idx_v.at[j0 + 2]], rows_v.at[0], sem0)

            pltpu.make_async_copy(
                table_hbm.at[idx_v.at[j0 + 1]], rows_v.at[1], sem1).wait()
            pltpu.sync_copy(rows_v.at[1],
                            out_hbm.at[pl.ds(base + (j0 + 1) * CH, CH)])

            @pl.when(jj < NCH // 2 - 1)
            def _():
                pltpu.async_copy(
                    table_hbm.at[idx_v.at[j0 + 3]], rows_v.at[1], sem1)

            return carry

        lax.fori_loop(0, NCH // 2, pair, 0)

    return k(idx, table)


def _gru_body(emb_ref, lens_ref, wrzf_ref, winf_ref, whnf_ref, brzf_ref,
              binf_ref, bhnf_ref, wrzb_ref, winb_ref, whnb_ref, brzb_ref,
              binb_ref, bhnb_ref, wcomb_ref, bl_ref, out_ref, hfb_ref):
    lens = lens_ref[...]
    linv = 1.0 / lens.astype(jnp.float32)
    half = jnp.bfloat16(0.5)

    def cell(x, h, h16, wrz_ref, win_ref, whn_ref, brz_ref, bin_ref, bhn_ref):
        xb = x.astype(bf16)
        xh = jnp.concatenate([xb, h16], axis=1)
        a = jnp.dot(xh, wrz_ref[...],
                    preferred_element_type=jnp.float32) + brz_ref[...]
        hn = jnp.dot(h16, whn_ref[...],
                     preferred_element_type=jnp.float32) + bhn_ref[...]
        i_n = jnp.dot(xb, win_ref[...],
                      preferred_element_type=jnp.float32) + bin_ref[...]
        tr = jnp.tanh(a[:, :H])
        tz = jnp.tanh(a[:, H:])
        n = jnp.tanh(i_n + hn + tr * hn)
        z = half * tz + half
        hnew = n + z * (h - n)
        return hnew, hnew.astype(bf16)

    h0 = jnp.zeros((B, H), jnp.float32)
    h016 = jnp.zeros((B, H), bf16)

    def cells(i, hf, hf16, hb, hb16):
        tb = L - 1 - i
        hf, hf16 = cell(emb_ref[i], hf, hf16, wrzf_ref, winf_ref, whnf_ref,
                        brzf_ref, binf_ref, bhnf_ref)
        hb, hb16 = cell(emb_ref[tb], hb, hb16, wrzb_ref, winb_ref, whnb_ref,
                        brzb_ref, binb_ref, bhnb_ref)
        hfb_ref[i, :, :H] = hf16
        hfb_ref[tb, :, H:] = hb16
        return hf, hf16, hb, hb16

    def comb(t, acc):
        y = jnp.dot(hfb_ref[t], wcomb_ref[...],
                    preferred_element_type=jnp.float32)
        y = jnp.maximum(y + bl_ref[...], 0.0)
        wc = jnp.sum(jnp.where(lens > t, linv, 0.0))
        return acc + wc * jnp.sum(y, axis=0, keepdims=True)

    def step_a(i, carry):
        return cells(i, *carry)

    carry = lax.fori_loop(0, L // 2, step_a, (h0, h016, h0, h016), unroll=25)

    def step_b(i, carry):
        hf, hf16, hb, hb16, acc = carry
        hf, hf16, hb, hb16 = cells(i, hf, hf16, hb, hb16)
        acc = comb(i, acc)
        acc = comb(L - 1 - i, acc)
        return hf, hf16, hb, hb16, acc

    acc0 = jnp.zeros((1, D), jnp.float32)
    carry = lax.fori_loop(L // 2, L, step_b, (*carry, acc0), unroll=25)
    out_ref[...] = carry[-1]


def _tc_gru(emb_lbd, lens_i, *weights):
    return pl.pallas_call(
        _gru_body,
        out_shape=jax.ShapeDtypeStruct((1, D), jnp.float32),
        scratch_shapes=[pltpu.VMEM((L, B, 2 * H), bf16)],
    )(emb_lbd, lens_i, *weights)


def _dir_weights(Wih, Whh, bih, bhh):
    wihT = Wih.T
    whhT = Whh.T
    wrz = (0.5 * jnp.concatenate([wihT[:, :2 * H], whhT[:, :2 * H]],
                                 axis=0)).astype(bf16)
    win = wihT[:, 2 * H:].astype(bf16)
    whn = (0.5 * whhT[:, 2 * H:]).astype(bf16)
    brz = (0.5 * (bih[:2 * H] + bhh[:2 * H])).reshape(1, 2 * H)
    bin_ = bih[2 * H:].reshape(1, H)
    bhn = (0.5 * bhh[2 * H:]).reshape(1, H)
    return wrz, win, whn, brz, bin_, bhn


def kernel(x_in, in_len, table, Wih_f, Whh_f, bih_f, bhh_f,
           Wih_b, Whh_b, bih_b, bhh_b, Wl, bl):
    x_in = x_in.astype(jnp.int32)
    idx = x_in.T.reshape(NW, NCH, CH)
    emb = _sc_gather(idx, table).reshape(L, B, D)

    lens_i = in_len.astype(jnp.int32).reshape(8, 128)
    out = _tc_gru(
        emb, lens_i,
        *_dir_weights(Wih_f, Whh_f, bih_f, bhh_f),
        *_dir_weights(Wih_b, Whh_b, bih_b, bhh_b),
        Wl.T.astype(bf16), bl.reshape(1, D),
    )
    return out
